# baseline (speedup 1.0000x reference)
import jax
import jax.numpy as jnp
from jax.experimental import pallas as pl
from jax.experimental.pallas import tpu as pltpu

B = 16; P = 16; NP = 128; EP = 1024
NPAT = 16; EPAT = 80
D_ORIG = 256; H1 = 64; HEADS = 2; OUT1 = 32; NH = 128
EPS = 1e-5
INV = 1.0 / (1.0 + EPS) ** 0.5
NEG = -1e30
NPAIR = 2
NPRG = P // NPAIR


def _lrelu(x):
    return jnp.maximum(x, 0.2 * x)


def _dot(a, b):
    return jax.lax.dot_general(a, b, (((1,), (0,)), ((), ())),
                               preferred_element_type=jnp.float32)


def _dot_t(a, b):
    return jax.lax.dot_general(a, b, (((1,), (1,)), ((), ())),
                               preferred_element_type=jnp.float32)


def _bdot(a, b, ca, cb):
    return jax.lax.dot_general(a, b, (((ca,), (cb,)), ((0,), (0,))),
                               preferred_element_type=jnp.float32)


def _fused_kernel(pf_col_ref, pf_row_ref, src_ref, dst_ref,
                  rows_ref, mats_ref, decw1_ref, decw2_ref, ftwp_ref,
                  orig_ref, ftw0_ref, prow_ref, gcw_ref, clw1_ref, clw2_ref,
                  psrc_ref, pdst_ref, mrow_ref, mcol_ref,
                  logits_ref, avg_ref,
                  ohs_s, ohd_s, lct_s, hacc_s, racc_s):
    p = pl.program_id(0)

    @pl.when(p == 0)
    def _init():
        iota_e = jax.lax.broadcasted_iota(jnp.int32, (NP, EP), 0)
        ohs = (src_ref[:, :] == iota_e).astype(jnp.float32)
        ohd = (dst_ref[:, :] == iota_e).astype(jnp.float32)
        ohs_s[:, :] = ohs
        ohd_s[:, :] = ohd
        ct = _dot_t(ohd, ohs)
        lct_s[:, :] = jnp.where(ct > 0.5, jnp.log(jnp.maximum(ct, 0.5)), NEG)
        hacc_s[:, :] = jnp.zeros((NPAT, NH), jnp.float32)
        racc_s[:, :] = jnp.zeros((1, 1), jnp.float32)

    ohs = ohs_s[:, :]
    ohd = ohd_s[:, :]
    lct3 = lct_s[:, :][None, :, :]

    for i in range(NPAIR):
        rows = rows_ref[i]
        fc1 = rows[0:1]; al1 = rows[1:2]; ar1 = rows[2:3]
        b1 = rows[4:5]; bn1g = rows[5:6]; bn1b = rows[6:7]
        db1 = rows[7:8]; dbng = rows[8:9]; dbnb = rows[9:10]
        db2 = rows_ref[i, 11, 0]
        g2al = rows[12:13, :OUT1]; g2ar = rows[13:14, :OUT1]
        g2b = rows[14:15, :OUT1]; bn2g = rows[15:16, :OUT1]
        bn2b = rows[16:17, :OUT1]
        g2fc = mats_ref[i][:, :OUT1]
        g2res = mats_ref[i][:, OUT1:]

        h0c = pf_col_ref[:, i, :, :]
        h0b = pf_row_ref[:, i, 0, :]

        prod_l = fc1 * al1
        prod_r = fc1 * ar1
        cl0 = jnp.sum(prod_l[:, :H1]); cl1 = jnp.sum(prod_l[:, H1:])
        cr0 = jnp.sum(prod_r[:, :H1]); cr1 = jnp.sum(prod_r[:, H1:])
        hs_e = _dot(h0b, ohs)

        def gat1_head(cl, cr):
            el = cl * h0b
            er = cr * h0b
            shift_n = _lrelu(jnp.max(el, axis=1, keepdims=True) + er)
            g = _dot(jnp.concatenate([er, shift_n], axis=0), ohd)
            ee = jnp.exp(_lrelu(cl * hs_e + g[:B]) - g[B:])
            z = _dot_t(jnp.concatenate([ee * hs_e, ee], axis=0), ohd)
            den = z[B:]
            return jnp.where(den > 0.0, z[:B] / den, 0.0)

        s0 = gat1_head(cl0, cr0)
        s1 = gat1_head(cl1, cr1)
        lane = jax.lax.broadcasted_iota(jnp.int32, (1, HEADS * H1), 1)
        hsel0 = (lane < H1).astype(jnp.float32)
        w3 = jnp.concatenate([fc1 * hsel0, fc1 * (1.0 - hsel0), rows[3:4]],
                             axis=0)
        lhs = jnp.concatenate(
            [s0.reshape(B, NP, 1), s1.reshape(B, NP, 1), h0c],
            axis=2).reshape(B * NP, 3)
        rst1 = _dot(lhs, w3) + b1
        h1f = jnp.maximum(rst1 * (INV * bn1g) + bn1b, 0.0)

        feat2 = _dot(h1f, g2fc).reshape(B, NP, OUT1)
        ones2 = feat2[:, :, 0:1] * 0.0 + 1.0
        feat2_aug = jnp.concatenate([feat2, ones2], axis=2)
        al2 = jnp.broadcast_to(g2al[None], (B, 1, OUT1))
        ar2 = jnp.broadcast_to(g2ar[None], (B, 1, OUT1))
        el2 = _bdot(al2, feat2, 2, 2)
        er2 = _bdot(feat2, ar2, 2, 2)
        shift2 = _lrelu(jnp.max(el2, axis=2, keepdims=True) + er2)
        ee2 = jnp.exp(_lrelu(el2 + er2) + lct3 - shift2)
        sums2 = _bdot(ee2, feat2_aug, 2, 1)
        den2 = sums2[:, :, OUT1:OUT1 + 1]
        rst2 = jnp.where(den2 > 0.0, sums2[:, :, :OUT1] / den2, 0.0)
        rst2 = rst2.reshape(B * NP, OUT1) + _dot(h1f, g2res) + g2b
        h2 = jnp.maximum(rst2 * (INV * bn2g) + bn2b, 0.0)

        rep = jnp.mean(h2.reshape(B, NP, OUT1), axis=1)
        hacc_s[:, :] += _dot(rep, ftwp_ref[i])

        d = _dot(h2, decw1_ref[i]) + db1
        d = jnp.maximum(d * (INV * dbng) + dbnb, 0.0)
        recon = _dot(d, decw2_ref[i]) + db2
        diff = recon.reshape(B, NP, 1) - h0c
        racc_s[:, :] += jnp.reshape(jnp.sum(diff * diff) / NP, (1, 1))

    @pl.when(p == NPRG - 1)
    def _patient():
        prow = prow_ref[:, :]
        h = _dot(orig_ref[:, :], ftw0_ref[:, :]) + hacc_s[:, :] + prow[0:1]
        h = jnp.maximum(h * (INV * prow[1:2]) + prow[2:3], 0.0)

        iota_e = jax.lax.broadcasted_iota(jnp.int32, (NPAT, EPAT), 0)
        pohs = (psrc_ref[:, :] == iota_e).astype(jnp.float32)
        pohd = (pdst_ref[:, :] == iota_e).astype(jnp.float32)
        cp = _dot_t(pohs, pohd)
        ctp = _dot_t(pohd, pohs)
        dout = jax.lax.rsqrt(jnp.maximum(jnp.sum(cp, axis=1, keepdims=True),
                                         1.0))
        din = jax.lax.rsqrt(jnp.maximum(jnp.sum(ctp, axis=1, keepdims=True),
                                        1.0))
        adj = ctp * mrow_ref[:, :] * mcol_ref[:, :]

        hsum = h
        for j in range(3):
            agg = _dot(adj, h * dout) * din
            hn = _dot(agg, gcw_ref[j]) + prow[3 + j:4 + j]
            hn = jnp.maximum(hn * (INV * prow[6 + j:7 + j])
                             + prow[9 + j:10 + j], 0.0)
            h = hn + h
            hsum = hsum + h
        havg = hsum * 0.25

        z = _dot(havg, clw1_ref[:, :]) + prow[12:13, :NH // 2]
        mu = jnp.mean(z, axis=1, keepdims=True)
        zc = z - mu
        var = jnp.mean(zc * zc, axis=1, keepdims=True)
        z = zc * jax.lax.rsqrt(var + EPS) * prow[13:14, :NH // 2] \
            + prow[14:15, :NH // 2]
        z = jnp.maximum(z, 0.0)
        logits_ref[:, :] = _dot(z, clw2_ref[:, :]) + prow[15:16, :2]
        avg_ref[:, :] = racc_s[:, :] * (1.0 / (B * P))


@jax.jit
def kernel(plane_feat, plane_edge_index, original_features, patient_edge_index,
           mask, g1_fc, g1_al, g1_ar, g1_res, g1_b, bn1_g, bn1_b,
           g2_fc, g2_al, g2_ar, g2_res, g2_b, bn2_g, bn2_b,
           dec_w1, dec_b1, dec_bng, dec_bnb, dec_w2, dec_b2,
           ft_w, ft_b, ft_bng, ft_bnb, gc_w, gc_b, gbn_g, gbn_b,
           cl_w1, cl_b1, cl_lng, cl_lnb, cl_w2, cl_b2):
    f32 = jnp.float32
    pf_col = plane_feat.astype(f32)
    pf_row = pf_col.reshape(B, P, 1, NP)
    src = plane_edge_index[0].astype(jnp.int32).reshape(1, EP)
    dst = plane_edge_index[1].astype(jnp.int32).reshape(1, EP)

    r1 = lambda a: a.reshape(P, 1, HEADS * H1)
    pad128 = lambda a: jnp.pad(a, ((0, 0), (0, 0), (0, NH - a.shape[2])))
    rows128 = jnp.concatenate([
        r1(g1_fc), r1(g1_al), r1(g1_ar), r1(g1_res), r1(g1_b),
        r1(bn1_g), r1(bn1_b), r1(dec_b1), r1(dec_bng), r1(dec_bnb),
        dec_w2.reshape(P, 1, NH),
        jnp.broadcast_to(dec_b2.reshape(P, 1, 1), (P, 1, NH)),
        pad128(g2_al), pad128(g2_ar), pad128(g2_b.reshape(P, 1, OUT1)),
        pad128(bn2_g.reshape(P, 1, OUT1)), pad128(bn2_b.reshape(P, 1, OUT1)),
        jnp.zeros((P, 7, NH), f32),
    ], axis=1)
    mats64 = jnp.concatenate([g2_fc, g2_res], axis=2)
    ftwp = ft_w[D_ORIG:].reshape(P, OUT1, NH)
    ftw0 = ft_w[:D_ORIG]

    pad_r = lambda a: jnp.pad(a.reshape(1, -1), ((0, 0), (0, NH - a.shape[0])))
    prow = jnp.concatenate([
        ft_b.reshape(1, NH), ft_bng.reshape(1, NH), ft_bnb.reshape(1, NH),
        gc_b, gbn_g, gbn_b,
        pad_r(cl_b1), pad_r(cl_lng), pad_r(cl_lnb), pad_r(cl_b2),
    ], axis=0)

    psrc = patient_edge_index[0].astype(jnp.int32).reshape(1, EPAT)
    pdst = patient_edge_index[1].astype(jnp.int32).reshape(1, EPAT)
    maskf = mask.astype(f32)

    pairspec = lambda blk: pl.BlockSpec(blk, lambda p: (p,) + (0,) * (len(blk) - 1))
    cspec = lambda blk: pl.BlockSpec(blk, lambda p: (0,) * len(blk))

    logits, avg = pl.pallas_call(
        _fused_kernel,
        grid=(NPRG,),
        in_specs=[
            pl.BlockSpec((B, NPAIR, NP, 1), lambda p: (0, p, 0, 0)),
            pl.BlockSpec((B, NPAIR, 1, NP), lambda p: (0, p, 0, 0)),
            cspec((1, EP)),
            cspec((1, EP)),
            pairspec((NPAIR, 24, NH)),
            pairspec((NPAIR, NH, 2 * OUT1)),
            pairspec((NPAIR, OUT1, NH)),
            pairspec((NPAIR, NH, 1)),
            pairspec((NPAIR, OUT1, NH)),
            cspec((NPAT, D_ORIG)),
            cspec((D_ORIG, NH)),
            cspec((16, NH)),
            cspec((3, NH, NH)),
            cspec((NH, NH // 2)),
            cspec((NH // 2, 2)),
            cspec((1, EPAT)),
            cspec((1, EPAT)),
            cspec((1, NPAT)),
            cspec((NPAT, 1)),
        ],
        out_specs=[
            pl.BlockSpec((NPAT, 2), lambda p: (0, 0)),
            pl.BlockSpec((1, 1), lambda p: (0, 0)),
        ],
        out_shape=[
            jax.ShapeDtypeStruct((NPAT, 2), f32),
            jax.ShapeDtypeStruct((1, 1), f32),
        ],
        scratch_shapes=[
            pltpu.VMEM((NP, EP), f32),
            pltpu.VMEM((NP, EP), f32),
            pltpu.VMEM((NP, NP), f32),
            pltpu.VMEM((NPAT, NH), f32),
            pltpu.VMEM((1, 1), f32),
        ],
        compiler_params=pltpu.CompilerParams(
            dimension_semantics=("arbitrary",)),
    )(pf_col, pf_row, src, dst, rows128, mats64, dec_w1, dec_w2, ftwp,
      original_features.astype(f32), ftw0, prow, gc_w, cl_w1, cl_w2,
      psrc, pdst, maskf.reshape(1, NPAT), maskf.reshape(NPAT, 1))

    return logits, avg.reshape(())


# speedup vs baseline: 1.1671x; 1.1671x over previous
import jax
import jax.numpy as jnp
from jax.experimental import pallas as pl
from jax.experimental.pallas import tpu as pltpu

B = 16; P = 16; NP = 128; EP = 1024
NPAT = 16; EPAT = 80
D_ORIG = 256; H1 = 64; HEADS = 2; OUT1 = 32; NH = 128
EPS = 1e-5
INV = 1.0 / (1.0 + EPS) ** 0.5
NEG = -1e30


def _lrelu(x):
    return jnp.maximum(x, 0.2 * x)


def _dot(a, b):
    return jax.lax.dot_general(a, b, (((1,), (0,)), ((), ())),
                               preferred_element_type=jnp.float32)


def _dot_t(a, b):
    return jax.lax.dot_general(a, b, (((1,), (1,)), ((), ())),
                               preferred_element_type=jnp.float32)


def _bdot(a, b, ca, cb):
    return jax.lax.dot_general(a, b, (((ca,), (cb,)), ((0,), (0,))),
                               preferred_element_type=jnp.float32)


def _ct_kernel(src_ref, dst_ref, lct_ref):
    iota_e = jax.lax.broadcasted_iota(jnp.int32, (NP, EP), 0)
    ohs = (src_ref[:, :] == iota_e).astype(jnp.float32)
    ohd = (dst_ref[:, :] == iota_e).astype(jnp.float32)
    ct = _dot_t(ohd, ohs)
    lct_ref[:, :] = jnp.where(ct > 0.5, jnp.log(jnp.maximum(ct, 0.5)), NEG)


def _plane_kernel(lct_ref, pf_col_ref, pf_row_ref,
                  g1_fc_ref, g1_al_ref, g1_ar_ref, g1_res_ref, g1_b_ref,
                  bn1_g_ref, bn1_b_ref,
                  g2_fc_ref, g2_al_ref, g2_ar_ref, g2_res_ref, g2_b_ref,
                  bn2_g_ref, bn2_b_ref,
                  dec_w1_ref, dec_b1_ref, dec_bng_ref, dec_bnb_ref,
                  dec_w2_ref, dec_b2_ref,
                  rep_ref, rloss_ref):
    lct3 = lct_ref[:, :][None, :, :]

    h0c = pf_col_ref[:, 0, :, :]
    h0r = pf_row_ref[:, 0, :, :]

    fc1 = g1_fc_ref[0]
    al1 = g1_al_ref[0]
    ar1 = g1_ar_ref[0]
    prod_l = fc1 * al1
    prod_r = fc1 * ar1
    cl0 = jnp.sum(prod_l[:, :H1]); cl1 = jnp.sum(prod_l[:, H1:])
    cr0 = jnp.sum(prod_r[:, :H1]); cr1 = jnp.sum(prod_r[:, H1:])
    hmax = jnp.max(h0r, axis=2, keepdims=True)
    hmin = jnp.min(h0r, axis=2, keepdims=True)

    def gat1_head(cl, cr):
        elmax = jnp.maximum(cl * hmax, cl * hmin)
        shift = _lrelu(elmax + cr * h0c)
        ee = jnp.exp(_lrelu(cl * h0r + cr * h0c) + lct3 - shift)
        den = jnp.sum(ee, axis=2, keepdims=True)
        num = jnp.sum(ee * h0r, axis=2, keepdims=True)
        return jnp.where(den > 0.0, num / den, 0.0)

    s0 = gat1_head(cl0, cr0)
    s1 = gat1_head(cl1, cr1)
    lane = jax.lax.broadcasted_iota(jnp.int32, (1, HEADS * H1), 1)
    hsel0 = (lane < H1).astype(jnp.float32)
    w3 = jnp.concatenate([fc1 * hsel0, fc1 * (1.0 - hsel0), g1_res_ref[0]],
                         axis=0)
    lhs = jnp.concatenate([s0, s1, h0c], axis=2).reshape(B * NP, 3)
    rst1 = _dot(lhs, w3) + g1_b_ref[0]
    h1f = jnp.maximum(rst1 * (INV * bn1_g_ref[0]) + bn1_b_ref[0], 0.0)

    feat2 = _dot(h1f, g2_fc_ref[0]).reshape(B, NP, OUT1)
    ones2 = feat2[:, :, 0:1] * 0.0 + 1.0
    feat2_aug = jnp.concatenate([feat2, ones2], axis=2)
    al2 = jnp.broadcast_to(g2_al_ref[0][None], (B, 1, OUT1))
    ar2 = jnp.broadcast_to(g2_ar_ref[0][None], (B, 1, OUT1))
    el2 = _bdot(al2, feat2, 2, 2)
    er2 = _bdot(feat2, ar2, 2, 2)
    shift2 = _lrelu(jnp.max(el2, axis=2, keepdims=True) + er2)
    ee2 = jnp.exp(_lrelu(el2 + er2) + lct3 - shift2)
    sums2 = _bdot(ee2, feat2_aug, 2, 1)
    den2 = sums2[:, :, OUT1:OUT1 + 1]
    rst2 = jnp.where(den2 > 0.0, sums2[:, :, :OUT1] / den2, 0.0)
    rst2 = (rst2.reshape(B * NP, OUT1) + _dot(h1f, g2_res_ref[0])
            + g2_b_ref[0])
    h2 = jnp.maximum(rst2 * (INV * bn2_g_ref[0]) + bn2_b_ref[0], 0.0)

    rep_ref[:, 0, :, :] = jnp.mean(h2.reshape(B, NP, OUT1), axis=1,
                                   keepdims=True)

    d = _dot(h2, dec_w1_ref[0]) + dec_b1_ref[0]
    d = jnp.maximum(d * (INV * dec_bng_ref[0]) + dec_bnb_ref[0], 0.0)
    recon = _dot(d, dec_w2_ref[0]) + dec_b2_ref[0, 0, 0]
    diff = recon.reshape(B, NP, 1) - h0c
    rloss_ref[:, 0, :, :] = jnp.sum(diff * diff, axis=(1, 2),
                                    keepdims=True) / NP


def _patient_kernel(nf_ref, psrc_ref, pdst_ref, mask_row_ref, mask_col_ref,
                    ft_w_ref, ft_b_ref, ft_bng_ref, ft_bnb_ref,
                    gc_w_ref, gc_b_ref, gbn_g_ref, gbn_b_ref,
                    cl_w1_ref, cl_b1_ref, cl_lng_ref, cl_lnb_ref,
                    cl_w2_ref, cl_b2_ref, rl_ref,
                    logits_ref, avg_ref):
    h = _dot(nf_ref[:, :], ft_w_ref[:, :]) + ft_b_ref[:, :]
    h = jnp.maximum(h * (INV * ft_bng_ref[:, :]) + ft_bnb_ref[:, :], 0.0)

    iota_e = jax.lax.broadcasted_iota(jnp.int32, (NPAT, EPAT), 0)
    ohs = (psrc_ref[:, :] == iota_e).astype(jnp.float32)
    ohd = (pdst_ref[:, :] == iota_e).astype(jnp.float32)
    cp = _dot_t(ohs, ohd)
    ctp = _dot_t(ohd, ohs)
    out_deg = jnp.maximum(jnp.sum(cp, axis=1, keepdims=True), 1.0)
    in_deg = jnp.maximum(jnp.sum(ctp, axis=1, keepdims=True), 1.0)
    dout = jax.lax.rsqrt(out_deg)
    din = jax.lax.rsqrt(in_deg)
    adj = ctp * mask_row_ref[:, :] * mask_col_ref[:, :]

    hsum = h
    for i in range(3):
        agg = _dot(adj, h * dout) * din
        hn = _dot(agg, gc_w_ref[i]) + gc_b_ref[i]
        hn = jnp.maximum(hn * (INV * gbn_g_ref[i]) + gbn_b_ref[i], 0.0)
        h = hn + h
        hsum = hsum + h
    havg = hsum * 0.25

    z = _dot(havg, cl_w1_ref[:, :]) + cl_b1_ref[:, :]
    mu = jnp.mean(z, axis=1, keepdims=True)
    zc = z - mu
    var = jnp.mean(zc * zc, axis=1, keepdims=True)
    z = zc * jax.lax.rsqrt(var + EPS) * cl_lng_ref[:, :] + cl_lnb_ref[:, :]
    z = jnp.maximum(z, 0.0)
    logits_ref[:, :] = _dot(z, cl_w2_ref[:, :]) + cl_b2_ref[:, :]
    avg_ref[:, :] = jnp.reshape(jnp.sum(rl_ref[:, :]) / (B * P), (1, 1))


@jax.jit
def kernel(plane_feat, plane_edge_index, original_features, patient_edge_index,
           mask, g1_fc, g1_al, g1_ar, g1_res, g1_b, bn1_g, bn1_b,
           g2_fc, g2_al, g2_ar, g2_res, g2_b, bn2_g, bn2_b,
           dec_w1, dec_b1, dec_bng, dec_bnb, dec_w2, dec_b2,
           ft_w, ft_b, ft_bng, ft_bnb, gc_w, gc_b, gbn_g, gbn_b,
           cl_w1, cl_b1, cl_lng, cl_lnb, cl_w2, cl_b2):
    f32 = jnp.float32
    pf_col = plane_feat.astype(f32)
    pf_row = pf_col.reshape(B, P, 1, NP)
    src = plane_edge_index[0].astype(jnp.int32).reshape(1, EP)
    dst = plane_edge_index[1].astype(jnp.int32).reshape(1, EP)

    lct = pl.pallas_call(
        _ct_kernel,
        out_shape=jax.ShapeDtypeStruct((NP, NP), f32),
    )(src, dst)

    pspec = lambda blk: pl.BlockSpec(blk, lambda p: (p,) + (0,) * (len(blk) - 1))
    cspec = lambda blk: pl.BlockSpec(blk, lambda p: (0,) * len(blk))
    bspec = lambda blk: pl.BlockSpec(blk, lambda p: (0, p) + (0,) * (len(blk) - 2))

    reps, rloss = pl.pallas_call(
        _plane_kernel,
        grid=(P,),
        in_specs=[
            cspec((NP, NP)),
            bspec((B, 1, NP, 1)),
            bspec((B, 1, 1, NP)),
            pspec((1, 1, HEADS * H1)),
            pspec((1, 1, HEADS * H1)),
            pspec((1, 1, HEADS * H1)),
            pspec((1, 1, HEADS * H1)),
            pspec((1, 1, HEADS * H1)),
            pspec((1, 1, HEADS * H1)),
            pspec((1, 1, HEADS * H1)),
            pspec((1, HEADS * H1, OUT1)),
            pspec((1, 1, OUT1)),
            pspec((1, 1, OUT1)),
            pspec((1, HEADS * H1, OUT1)),
            pspec((1, 1, OUT1)),
            pspec((1, 1, OUT1)),
            pspec((1, 1, OUT1)),
            pspec((1, OUT1, HEADS * H1)),
            pspec((1, 1, HEADS * H1)),
            pspec((1, 1, HEADS * H1)),
            pspec((1, 1, HEADS * H1)),
            pspec((1, HEADS * H1, 1)),
            pspec((1, 1, 1)),
        ],
        out_specs=[
            pl.BlockSpec((B, 1, 1, OUT1), lambda p: (0, p, 0, 0)),
            pl.BlockSpec((B, 1, 1, 1), lambda p: (0, p, 0, 0)),
        ],
        out_shape=[
            jax.ShapeDtypeStruct((B, P, 1, OUT1), f32),
            jax.ShapeDtypeStruct((B, P, 1, 1), f32),
        ],
        compiler_params=pltpu.CompilerParams(
            dimension_semantics=("arbitrary",)),
    )(lct, pf_col, pf_row,
      g1_fc.reshape(P, 1, HEADS * H1), g1_al.reshape(P, 1, HEADS * H1),
      g1_ar.reshape(P, 1, HEADS * H1), g1_res.reshape(P, 1, HEADS * H1),
      g1_b.reshape(P, 1, HEADS * H1), bn1_g.reshape(P, 1, HEADS * H1),
      bn1_b.reshape(P, 1, HEADS * H1),
      g2_fc, g2_al, g2_ar, g2_res,
      g2_b.reshape(P, 1, OUT1), bn2_g.reshape(P, 1, OUT1),
      bn2_b.reshape(P, 1, OUT1),
      dec_w1, dec_b1.reshape(P, 1, HEADS * H1),
      dec_bng.reshape(P, 1, HEADS * H1), dec_bnb.reshape(P, 1, HEADS * H1),
      dec_w2, dec_b2.reshape(P, 1, 1))

    node_features = jnp.concatenate(
        [original_features.astype(f32), reps.reshape(B, P * OUT1)], axis=1)
    psrc = patient_edge_index[0].astype(jnp.int32).reshape(1, EPAT)
    pdst = patient_edge_index[1].astype(jnp.int32).reshape(1, EPAT)
    maskf = mask.astype(f32)

    logits, avg = pl.pallas_call(
        _patient_kernel,
        out_shape=[
            jax.ShapeDtypeStruct((NPAT, 2), f32),
            jax.ShapeDtypeStruct((1, 1), f32),
        ],
    )(node_features, psrc, pdst, maskf.reshape(1, NPAT),
      maskf.reshape(NPAT, 1),
      ft_w, ft_b.reshape(1, NH), ft_bng.reshape(1, NH), ft_bnb.reshape(1, NH),
      gc_w, gc_b.reshape(3, 1, NH), gbn_g.reshape(3, 1, NH),
      gbn_b.reshape(3, 1, NH),
      cl_w1, cl_b1.reshape(1, NH // 2), cl_lng.reshape(1, NH // 2),
      cl_lnb.reshape(1, NH // 2), cl_w2, cl_b2.reshape(1, 2),
      rloss.reshape(B, P))

    return logits, avg.reshape(())
